# baseline (speedup 1.0000x reference)
import functools

import jax
import jax.numpy as jnp
from jax import lax
from jax.experimental import pallas as pl
from jax.experimental.pallas import tpu as pltpu
from jax.experimental.pallas import tpu_sc as plsc

N = 10000
E = 160000
D = 256
HALF = 128

NP = 10240
NTILES = 16
NC = 2
RPT = NP // NTILES
CH = 128
NCHUNK = NP // CH
NROW = NP // 128
ED_CH = 40
ED_PAD = ED_CH * CH
G = 16
NGRP = NCHUNK // G

BR = 1024
GR = 2000
GC = 1280


def _deg_body(dst_hbm, out_hbm, dst_v, hist, sem):
    cid = lax.axis_index("c")
    sid = lax.axis_index("s")
    wid = sid * NC + cid
    pltpu.async_copy(dst_hbm.at[wid], dst_v, sem).wait()
    zeros = jnp.zeros((16,), jnp.float32)

    @pl.loop(0, NROW)
    def _(i):
        @pl.loop(0, 8)
        def _(j):
            hist[i, pl.ds(j * 16, 16)] = zeros

    ones = jnp.ones((16,), jnp.float32)

    @pl.loop(0, ED_CH)
    def _(i):
        @pl.loop(0, 8)
        def _(j):
            idx = dst_v[i, pl.ds(j * 16, 16)]
            plsc.addupdate_scatter(hist, [idx >> 7, idx & 127], ones)

    pltpu.sync_copy(hist, out_hbm.at[wid])


def _make_deg_kernel():
    mesh = plsc.VectorSubcoreMesh(core_axis_name="c", subcore_axis_name="s")
    return pl.kernel(
        _deg_body,
        out_type=jax.ShapeDtypeStruct((NC * NTILES, NROW, 128), jnp.float32),
        mesh=mesh,
        compiler_params=pltpu.CompilerParams(needs_layout_passes=False),
        scratch_types=[
            pltpu.VMEM((ED_CH, CH), jnp.int32),
            pltpu.VMEM((NROW, 128), jnp.float32),
            pltpu.SemaphoreType.DMA,
        ],
    )


def _conv_body(y2, src_hbm, dst_hbm, out_hbm, src_v, dst_v, rows,
               acc, sem_i, gs0, gs1, ss0, ss1):
    gsem = (gs0, gs1)
    ssem = (ss0, ss1)
    cid = lax.axis_index("c")
    sid = lax.axis_index("s")
    base = sid * RPT

    def idx_load(grp, wait):
        p = grp & 1
        s_src = src_hbm.at[cid, sid, pl.ds(grp * G, G)]
        s_dst = dst_hbm.at[sid, pl.ds(grp * G, G)]
        if wait:
            pltpu.make_async_copy(s_src, src_v.at[p], sem_i).wait()
            pltpu.make_async_copy(s_dst, dst_v.at[p], sem_i).wait()
        else:
            pltpu.async_copy(s_src, src_v.at[p], sem_i)
            pltpu.async_copy(s_dst, dst_v.at[p], sem_i)

    def iref(v, k):
        return v.at[(k // G) & 1, k % G]

    def gather(k, b):
        pltpu.async_copy(y2.at[iref(src_v, k)], rows.at[b], gsem[b])

    def gwait(k, b):
        pltpu.make_async_copy(y2.at[iref(src_v, k)], rows.at[b],
                              gsem[b]).wait()

    def scatter(k, b):
        pltpu.async_copy(rows.at[b], acc.at[iref(dst_v, k)],
                         ssem[b], add=True)

    def swait(k, b):
        pltpu.make_async_copy(rows.at[b], acc.at[iref(dst_v, k)],
                              ssem[b]).wait()

    idx_load(0, False)
    pltpu.sync_copy(y2.at[pl.ds(cid * NP + base, RPT)],
                    acc.at[pl.ds(base, RPT)])
    idx_load(0, True)
    plsc.subcore_barrier()

    for grp in range(NGRP):
        for j in range(G):
            k = grp * G + j
            b = j & 1
            if k >= 2:
                gwait(k - 2, b)
                scatter(k - 2, b)
                swait(k - 2, b)
            gather(k, b)
            if j == 4 and grp + 1 < NGRP:
                idx_load(grp + 1, False)
            if j == G - 1 and grp + 1 < NGRP:
                idx_load(grp + 1, True)

    for j in range(2):
        k = NCHUNK - 2 + j
        gwait(k, k & 1)
        scatter(k, k & 1)
        swait(k, k & 1)

    plsc.subcore_barrier()
    pltpu.sync_copy(acc.at[pl.ds(base, RPT)],
                    out_hbm.at[pl.ds(cid * NP + base, RPT)])


def _make_conv_kernel():
    mesh = plsc.VectorSubcoreMesh(core_axis_name="c", subcore_axis_name="s")
    return pl.kernel(
        _conv_body,
        out_type=jax.ShapeDtypeStruct((NC * NP, HALF), jnp.float32),
        mesh=mesh,
        compiler_params=pltpu.CompilerParams(needs_layout_passes=False),
        scratch_types=[
            pltpu.VMEM((2, G, CH), jnp.int32),
            pltpu.VMEM((2, G, CH), jnp.int32),
            pltpu.VMEM((2, CH, HALF), jnp.float32),
            pltpu.VMEM_SHARED((NP, HALF), jnp.float32),
            pltpu.SemaphoreType.DMA,
            pltpu.SemaphoreType.DMA,
            pltpu.SemaphoreType.DMA,
            pltpu.SemaphoreType.DMA,
            pltpu.SemaphoreType.DMA,
        ],
    )


def _split(y):
    return jnp.stack([y[:, :HALF], y[:, HALF:]], axis=0)


def _cat(a):
    return jnp.concatenate([a[0], a[1]], axis=1)


def _prep_body(hist_ref, x_ref, w_ref, dinvb_ref, y_ref):
    deg = jnp.sum(hist_ref[...], axis=1, keepdims=True) + 1.0
    dvb = jnp.broadcast_to(lax.rsqrt(deg), (BR, D))
    dinvb_ref[...] = dvb
    y = jnp.dot(x_ref[...], w_ref[...], preferred_element_type=jnp.float32)
    y_ref[...] = _split(y * dvb)


def _prep_call(hist, x, w1):
    return pl.pallas_call(
        _prep_body,
        grid=(NP // BR,),
        in_specs=[
            pl.BlockSpec((BR, NC * NTILES), lambda i: (i, 0)),
            pl.BlockSpec((BR, D), lambda i: (i, 0)),
            pl.BlockSpec((D, D), lambda i: (0, 0)),
        ],
        out_specs=[
            pl.BlockSpec((BR, D), lambda i: (i, 0)),
            pl.BlockSpec((2, BR, HALF), lambda i: (0, i, 0)),
        ],
        out_shape=[
            jax.ShapeDtypeStruct((NP, D), jnp.float32),
            jax.ShapeDtypeStruct((2, NP, HALF), jnp.float32),
        ],
    )(hist, x, w1)


def _conv_tc_body(relu, nw, acc_ref, dinvb_ref, b_ref, *w_and_out):
    w_refs = w_and_out[:nw]
    out_refs = w_and_out[nw:]
    z = _cat(acc_ref[...])
    dvb = dinvb_ref[...]
    z = z * dvb + b_ref[...]
    if relu:
        z = jnp.maximum(z, 0.0)
    for w_ref, out_ref in zip(w_refs, out_refs):
        y = jnp.dot(z, w_ref[...], preferred_element_type=jnp.float32)
        out_ref[...] = _split(y * dvb)


def _conv_tc_call(acc, dinvb, b, ws, relu):
    nw = len(ws)
    return pl.pallas_call(
        functools.partial(_conv_tc_body, relu, nw),
        grid=(NP // BR,),
        in_specs=[
            pl.BlockSpec((2, BR, HALF), lambda i: (0, i, 0)),
            pl.BlockSpec((BR, D), lambda i: (i, 0)),
            pl.BlockSpec((1, D), lambda i: (0, 0)),
        ] + [pl.BlockSpec((D, D), lambda i: (0, 0))] * nw,
        out_specs=[pl.BlockSpec((2, BR, HALF), lambda i: (0, i, 0))] * nw,
        out_shape=[jax.ShapeDtypeStruct((2, NP, HALF), jnp.float32)] * nw,
    )(acc, dinvb, b, *ws)


def _final_body(acc4_ref, acc5_ref, dinvb_ref, ba_ref, bs_ref,
                x_ref, h_ref):
    dvb = dinvb_ref[...]
    x_ref[...] = _cat(acc4_ref[...]) * dvb + ba_ref[...]
    h_ref[...] = _cat(acc5_ref[...]) * dvb + bs_ref[...]


def _final_call(acc4, acc5, dinvb, ba, bs):
    return pl.pallas_call(
        _final_body,
        grid=(NP // BR,),
        in_specs=[
            pl.BlockSpec((2, BR, HALF), lambda i: (0, i, 0)),
            pl.BlockSpec((2, BR, HALF), lambda i: (0, i, 0)),
            pl.BlockSpec((BR, D), lambda i: (i, 0)),
            pl.BlockSpec((1, D), lambda i: (0, 0)),
            pl.BlockSpec((1, D), lambda i: (0, 0)),
        ],
        out_specs=[
            pl.BlockSpec((BR, D), lambda i: (i, 0)),
            pl.BlockSpec((BR, D), lambda i: (i, 0)),
        ],
        out_shape=[
            jax.ShapeDtypeStruct((NP, D), jnp.float32),
            jax.ShapeDtypeStruct((NP, D), jnp.float32),
        ],
    )(acc4, acc5, dinvb, ba, bs)


def _gram_body(a_ref, b_ref, out_ref):
    out_ref[0] = lax.dot_general(
        a_ref[...], b_ref[...], (((1,), (1,)), ((), ())),
        preferred_element_type=jnp.float32)


def _gram_call(h):
    out = pl.pallas_call(
        _gram_body,
        grid=(N // GR, pl.cdiv(N, GC)),
        in_specs=[
            pl.BlockSpec((GR, D), lambda i, j: (i, 0)),
            pl.BlockSpec((GC, D), lambda i, j: (j, 0)),
        ],
        out_specs=pl.BlockSpec((1, GR, GC), lambda i, j: (i, 0, j)),
        out_shape=jax.ShapeDtypeStruct((N // GR, GR, N), jnp.float32),
    )(h, h)
    return out.reshape(N, N)


def _pad_edges(idx, tiles, chunks, fill):
    per = chunks * CH
    take = E // tiles
    t = idx.reshape(tiles, take)
    pad = jnp.full((tiles, per - take), fill, jnp.int32)
    return jnp.concatenate([t, pad], axis=1).reshape(tiles, chunks, CH)


def kernel(x, edge_index, enc_W1, enc_b1, enc_W2, enc_b2,
           attr_W1, attr_b1, attr_W2, attr_b2, str_W1, str_b1):
    src = edge_index[0].astype(jnp.int32)
    dst = edge_index[1].astype(jnp.int32)

    src_p = _pad_edges(src, NTILES, NCHUNK, 0)
    src_all = jnp.stack([src_p, src_p + NP], axis=0)
    dst_p = _pad_edges(dst, NTILES, NCHUNK, N)
    dst_d = _pad_edges(dst, NC * NTILES, ED_CH, N)
    x_p = jnp.concatenate([x, jnp.zeros((NP - N, D), jnp.float32)], axis=0)

    deg_k = _make_deg_kernel()
    conv_k = _make_conv_kernel()

    hist = deg_k(dst_d)
    hist_n = hist.reshape(NC * NTILES, NP).T
    dinvb, y1 = _prep_call(hist_n, x_p, enc_W1)

    def conv(y):
        acc = conv_k(y.reshape(NC * NP, HALF), src_all, dst_p)
        return acc.reshape(2, NP, HALF)

    b = lambda v: v.reshape(1, D)

    acc1 = conv(y1)
    (y2,) = _conv_tc_call(acc1, dinvb, b(enc_b1), [enc_W2], relu=True)
    acc2 = conv(y2)
    y3, y5 = _conv_tc_call(acc2, dinvb, b(enc_b2), [attr_W1, str_W1],
                           relu=False)
    acc3 = conv(y3)
    (y4,) = _conv_tc_call(acc3, dinvb, b(attr_b1), [attr_W2], relu=True)
    acc4 = conv(y4)
    acc5 = conv(y5)
    x_full, h_full = _final_call(acc4, acc5, dinvb, b(attr_b2), b(str_b1))
    x_ = x_full[:N]
    s_ = _gram_call(h_full[:N])
    return (x_, s_)


# speedup vs baseline: 1.0095x; 1.0095x over previous
import functools

import jax
import jax.numpy as jnp
from jax import lax
from jax.experimental import pallas as pl
from jax.experimental.pallas import tpu as pltpu
from jax.experimental.pallas import tpu_sc as plsc

N = 10000
E = 160000
D = 256
HALF = 128

NP = 10240
NTILES = 16
NC = 2
RPT = NP // NTILES
CH = 128
CCH = 64
NBUF = 4
NCHUNK = NP // CCH
NROW = NP // 128
ED_CH = 40
ED_PAD = ED_CH * CH
G = 16
NGRP = NCHUNK // G

BR = 1024
GR = 2000
GC = 1280


def _deg_body(dst_hbm, out_hbm, dst_v, hist, sem):
    cid = lax.axis_index("c")
    sid = lax.axis_index("s")
    wid = sid * NC + cid
    pltpu.async_copy(dst_hbm.at[wid], dst_v, sem).wait()
    zeros = jnp.zeros((16,), jnp.float32)

    @pl.loop(0, NROW)
    def _(i):
        @pl.loop(0, 8)
        def _(j):
            hist[i, pl.ds(j * 16, 16)] = zeros

    ones = jnp.ones((16,), jnp.float32)

    @pl.loop(0, ED_CH)
    def _(i):
        @pl.loop(0, 8)
        def _(j):
            idx = dst_v[i, pl.ds(j * 16, 16)]
            plsc.addupdate_scatter(hist, [idx >> 7, idx & 127], ones)

    pltpu.sync_copy(hist, out_hbm.at[wid])


def _make_deg_kernel():
    mesh = plsc.VectorSubcoreMesh(core_axis_name="c", subcore_axis_name="s")
    return pl.kernel(
        _deg_body,
        out_type=jax.ShapeDtypeStruct((NC * NTILES, NROW, 128), jnp.float32),
        mesh=mesh,
        compiler_params=pltpu.CompilerParams(needs_layout_passes=False),
        scratch_types=[
            pltpu.VMEM((ED_CH, CH), jnp.int32),
            pltpu.VMEM((NROW, 128), jnp.float32),
            pltpu.SemaphoreType.DMA,
        ],
    )


def _conv_body(y2, src_hbm, dst_hbm, out_hbm, src_v, dst_v, rows,
               acc, sem_i, *sems):
    gsem = sems[:NBUF]
    ssem = sems[NBUF:]
    cid = lax.axis_index("c")
    sid = lax.axis_index("s")
    base = sid * RPT

    def idx_load(grp, wait):
        p = grp & 1
        s_src = src_hbm.at[cid, sid, pl.ds(grp * G, G)]
        s_dst = dst_hbm.at[sid, pl.ds(grp * G, G)]
        if wait:
            pltpu.make_async_copy(s_src, src_v.at[p], sem_i).wait()
            pltpu.make_async_copy(s_dst, dst_v.at[p], sem_i).wait()
        else:
            pltpu.async_copy(s_src, src_v.at[p], sem_i)
            pltpu.async_copy(s_dst, dst_v.at[p], sem_i)

    def iref(v, k):
        return v.at[(k // G) & 1, k % G]

    def gather(k, b):
        pltpu.async_copy(y2.at[iref(src_v, k)], rows.at[b], gsem[b])

    def gwait(k, b):
        pltpu.make_async_copy(y2.at[iref(src_v, k)], rows.at[b],
                              gsem[b]).wait()

    def scatter(k, b):
        pltpu.async_copy(rows.at[b], acc.at[iref(dst_v, k)],
                         ssem[b], add=True)

    def swait(k, b):
        pltpu.make_async_copy(rows.at[b], acc.at[iref(dst_v, k)],
                              ssem[b]).wait()

    idx_load(0, False)
    pltpu.sync_copy(y2.at[pl.ds(cid * NP + base, RPT)],
                    acc.at[pl.ds(base, RPT)])
    idx_load(0, True)
    plsc.subcore_barrier()

    for grp in range(NGRP):
        for j in range(G):
            k = grp * G + j
            if k >= NBUF:
                swait(k - NBUF, (k - NBUF) % NBUF)
            gather(k, k % NBUF)
            if k >= 2:
                gwait(k - 2, (k - 2) % NBUF)
                scatter(k - 2, (k - 2) % NBUF)
            if j == 4 and grp + 1 < NGRP:
                idx_load(grp + 1, False)
            if j == G - 1 and grp + 1 < NGRP:
                idx_load(grp + 1, True)

    for j in range(2):
        k = NCHUNK - 2 + j
        gwait(k, k % NBUF)
        scatter(k, k % NBUF)
    for j in range(NBUF):
        k = NCHUNK - NBUF + j
        swait(k, k % NBUF)

    plsc.subcore_barrier()
    pltpu.sync_copy(acc.at[pl.ds(base, RPT)],
                    out_hbm.at[pl.ds(cid * NP + base, RPT)])


def _make_conv_kernel():
    mesh = plsc.VectorSubcoreMesh(core_axis_name="c", subcore_axis_name="s")
    return pl.kernel(
        _conv_body,
        out_type=jax.ShapeDtypeStruct((NC * NP, HALF), jnp.float32),
        mesh=mesh,
        compiler_params=pltpu.CompilerParams(needs_layout_passes=False),
        scratch_types=[
            pltpu.VMEM((2, G, CCH), jnp.int32),
            pltpu.VMEM((2, G, CCH), jnp.int32),
            pltpu.VMEM((NBUF, CCH, HALF), jnp.float32),
            pltpu.VMEM_SHARED((NP, HALF), jnp.float32),
            pltpu.SemaphoreType.DMA,
        ] + [pltpu.SemaphoreType.DMA] * (2 * NBUF),
    )


def _split(y):
    return jnp.stack([y[:, :HALF], y[:, HALF:]], axis=0)


def _cat(a):
    return jnp.concatenate([a[0], a[1]], axis=1)


def _prep_body(hist_ref, x_ref, w_ref, dinvb_ref, y_ref):
    deg = jnp.sum(hist_ref[...], axis=1, keepdims=True) + 1.0
    dvb = jnp.broadcast_to(lax.rsqrt(deg), (BR, D))
    dinvb_ref[...] = dvb
    y = jnp.dot(x_ref[...], w_ref[...], preferred_element_type=jnp.float32)
    y_ref[...] = _split(y * dvb)


def _prep_call(hist, x, w1):
    return pl.pallas_call(
        _prep_body,
        grid=(NP // BR,),
        in_specs=[
            pl.BlockSpec((BR, NC * NTILES), lambda i: (i, 0)),
            pl.BlockSpec((BR, D), lambda i: (i, 0)),
            pl.BlockSpec((D, D), lambda i: (0, 0)),
        ],
        out_specs=[
            pl.BlockSpec((BR, D), lambda i: (i, 0)),
            pl.BlockSpec((2, BR, HALF), lambda i: (0, i, 0)),
        ],
        out_shape=[
            jax.ShapeDtypeStruct((NP, D), jnp.float32),
            jax.ShapeDtypeStruct((2, NP, HALF), jnp.float32),
        ],
    )(hist, x, w1)


def _conv_tc_body(relu, nw, acc_ref, dinvb_ref, b_ref, *w_and_out):
    w_refs = w_and_out[:nw]
    out_refs = w_and_out[nw:]
    z = _cat(acc_ref[...])
    dvb = dinvb_ref[...]
    z = z * dvb + b_ref[...]
    if relu:
        z = jnp.maximum(z, 0.0)
    for w_ref, out_ref in zip(w_refs, out_refs):
        y = jnp.dot(z, w_ref[...], preferred_element_type=jnp.float32)
        out_ref[...] = _split(y * dvb)


def _conv_tc_call(acc, dinvb, b, ws, relu):
    nw = len(ws)
    return pl.pallas_call(
        functools.partial(_conv_tc_body, relu, nw),
        grid=(NP // BR,),
        in_specs=[
            pl.BlockSpec((2, BR, HALF), lambda i: (0, i, 0)),
            pl.BlockSpec((BR, D), lambda i: (i, 0)),
            pl.BlockSpec((1, D), lambda i: (0, 0)),
        ] + [pl.BlockSpec((D, D), lambda i: (0, 0))] * nw,
        out_specs=[pl.BlockSpec((2, BR, HALF), lambda i: (0, i, 0))] * nw,
        out_shape=[jax.ShapeDtypeStruct((2, NP, HALF), jnp.float32)] * nw,
    )(acc, dinvb, b, *ws)


def _final_body(acc4_ref, acc5_ref, dinvb_ref, ba_ref, bs_ref,
                x_ref, h_ref):
    dvb = dinvb_ref[...]
    x_ref[...] = _cat(acc4_ref[...]) * dvb + ba_ref[...]
    h_ref[...] = _cat(acc5_ref[...]) * dvb + bs_ref[...]


def _final_call(acc4, acc5, dinvb, ba, bs):
    return pl.pallas_call(
        _final_body,
        grid=(NP // BR,),
        in_specs=[
            pl.BlockSpec((2, BR, HALF), lambda i: (0, i, 0)),
            pl.BlockSpec((2, BR, HALF), lambda i: (0, i, 0)),
            pl.BlockSpec((BR, D), lambda i: (i, 0)),
            pl.BlockSpec((1, D), lambda i: (0, 0)),
            pl.BlockSpec((1, D), lambda i: (0, 0)),
        ],
        out_specs=[
            pl.BlockSpec((BR, D), lambda i: (i, 0)),
            pl.BlockSpec((BR, D), lambda i: (i, 0)),
        ],
        out_shape=[
            jax.ShapeDtypeStruct((NP, D), jnp.float32),
            jax.ShapeDtypeStruct((NP, D), jnp.float32),
        ],
    )(acc4, acc5, dinvb, ba, bs)


def _gram_body(a_ref, b_ref, out_ref):
    out_ref[0] = lax.dot_general(
        a_ref[...], b_ref[...], (((1,), (1,)), ((), ())),
        preferred_element_type=jnp.float32)


def _gram_call(h):
    out = pl.pallas_call(
        _gram_body,
        grid=(N // GR, pl.cdiv(N, GC)),
        in_specs=[
            pl.BlockSpec((GR, D), lambda i, j: (i, 0)),
            pl.BlockSpec((GC, D), lambda i, j: (j, 0)),
        ],
        out_specs=pl.BlockSpec((1, GR, GC), lambda i, j: (i, 0, j)),
        out_shape=jax.ShapeDtypeStruct((N // GR, GR, N), jnp.float32),
    )(h, h)
    return out.reshape(N, N)


def _pad_edges(idx, tiles, chunks, width, fill):
    per = chunks * width
    take = E // tiles
    t = idx.reshape(tiles, take)
    pad = jnp.full((tiles, per - take), fill, jnp.int32)
    return jnp.concatenate([t, pad], axis=1).reshape(tiles, chunks, width)


def kernel(x, edge_index, enc_W1, enc_b1, enc_W2, enc_b2,
           attr_W1, attr_b1, attr_W2, attr_b2, str_W1, str_b1):
    src = edge_index[0].astype(jnp.int32)
    dst = edge_index[1].astype(jnp.int32)

    src_p = _pad_edges(src, NTILES, NCHUNK, CCH, 0)
    src_all = jnp.stack([src_p, src_p + NP], axis=0)
    dst_p = _pad_edges(dst, NTILES, NCHUNK, CCH, N)
    dst_d = _pad_edges(dst, NC * NTILES, ED_CH, CH, N)
    x_p = jnp.concatenate([x, jnp.zeros((NP - N, D), jnp.float32)], axis=0)

    deg_k = _make_deg_kernel()
    conv_k = _make_conv_kernel()

    hist = deg_k(dst_d)
    hist_n = hist.reshape(NC * NTILES, NP).T
    dinvb, y1 = _prep_call(hist_n, x_p, enc_W1)

    def conv(y):
        acc = conv_k(y.reshape(NC * NP, HALF), src_all, dst_p)
        return acc.reshape(2, NP, HALF)

    b = lambda v: v.reshape(1, D)

    acc1 = conv(y1)
    (y2,) = _conv_tc_call(acc1, dinvb, b(enc_b1), [enc_W2], relu=True)
    acc2 = conv(y2)
    y3, y5 = _conv_tc_call(acc2, dinvb, b(enc_b2), [attr_W1, str_W1],
                           relu=False)
    acc3 = conv(y3)
    (y4,) = _conv_tc_call(acc3, dinvb, b(attr_b1), [attr_W2], relu=True)
    acc4 = conv(y4)
    acc5 = conv(y5)
    x_full, h_full = _final_call(acc4, acc5, dinvb, b(attr_b2), b(str_b1))
    x_ = x_full[:N]
    s_ = _gram_call(h_full[:N])
    return (x_, s_)


# speedup vs baseline: 1.0267x; 1.0171x over previous
import functools

import jax
import jax.numpy as jnp
from jax import lax
from jax.experimental import pallas as pl
from jax.experimental.pallas import tpu as pltpu
from jax.experimental.pallas import tpu_sc as plsc

N = 10000
E = 160000
D = 256
HALF = 128

NP = 10240
NTILES = 16
NC = 2
RPT = NP // NTILES
CH = 128
CCH = 64
NBUF = 4
NCHUNK = NP // CCH
NROW = NP // 128
ED_CH = 40
ED_PAD = ED_CH * CH
G = 16
NGRP = NCHUNK // G

BR = 1024
GR = 2000
GC = 1280


def _deg_body(dst_hbm, out_hbm, dst_v, hist, sem):
    cid = lax.axis_index("c")
    sid = lax.axis_index("s")
    wid = sid * NC + cid
    pltpu.async_copy(dst_hbm.at[wid], dst_v, sem).wait()
    zeros = jnp.zeros((16,), jnp.float32)

    @pl.loop(0, NROW)
    def _(i):
        @pl.loop(0, 8)
        def _(j):
            hist[i, pl.ds(j * 16, 16)] = zeros

    ones = jnp.ones((16,), jnp.float32)

    @pl.loop(0, ED_CH)
    def _(i):
        @pl.loop(0, 8)
        def _(j):
            idx = dst_v[i, pl.ds(j * 16, 16)]
            plsc.addupdate_scatter(hist, [idx >> 7, idx & 127], ones)

    pltpu.sync_copy(hist, out_hbm.at[wid])


def _make_deg_kernel():
    mesh = plsc.VectorSubcoreMesh(core_axis_name="c", subcore_axis_name="s")
    return pl.kernel(
        _deg_body,
        out_type=jax.ShapeDtypeStruct((NC * NTILES, NROW, 128), jnp.float32),
        mesh=mesh,
        compiler_params=pltpu.CompilerParams(needs_layout_passes=False),
        scratch_types=[
            pltpu.VMEM((ED_CH, CH), jnp.int32),
            pltpu.VMEM((NROW, 128), jnp.float32),
            pltpu.SemaphoreType.DMA,
        ],
    )


def _conv_body(y2, src_hbm, dst_hbm, out_hbm, src_v, dst_v, rows,
               acc, sem_i, *sems):
    gsem = sems[:NBUF]
    ssem = sems[NBUF:]
    cid = lax.axis_index("c")
    sid = lax.axis_index("s")
    base = sid * RPT

    def idx_load(grp, wait):
        p = grp & 1
        s_src = src_hbm.at[cid, sid, pl.ds(grp * G, G)]
        s_dst = dst_hbm.at[sid, pl.ds(grp * G, G)]
        if wait:
            pltpu.make_async_copy(s_src, src_v.at[p], sem_i).wait()
            pltpu.make_async_copy(s_dst, dst_v.at[p], sem_i).wait()
        else:
            pltpu.async_copy(s_src, src_v.at[p], sem_i)
            pltpu.async_copy(s_dst, dst_v.at[p], sem_i)

    def iref(v, k):
        return v.at[(k // G) & 1, k % G]

    def gather(k, b):
        pltpu.async_copy(y2.at[iref(src_v, k)], rows.at[b], gsem[b])

    def gwait(k, b):
        pltpu.make_async_copy(y2.at[iref(src_v, k)], rows.at[b],
                              gsem[b]).wait()

    def scatter(k, b):
        pltpu.async_copy(rows.at[b], acc.at[iref(dst_v, k)],
                         ssem[b], add=True)

    def swait(k, b):
        pltpu.make_async_copy(rows.at[b], acc.at[iref(dst_v, k)],
                              ssem[b]).wait()

    idx_load(0, False)
    pltpu.sync_copy(y2.at[pl.ds(cid * NP + base, RPT)],
                    acc.at[pl.ds(base, RPT)])
    idx_load(0, True)
    plsc.subcore_barrier()

    for grp in range(NGRP):
        for j in range(G):
            k = grp * G + j
            if k >= NBUF:
                swait(k - NBUF, (k - NBUF) % NBUF)
            gather(k, k % NBUF)
            if k >= 2:
                gwait(k - 2, (k - 2) % NBUF)
                scatter(k - 2, (k - 2) % NBUF)
            if j == 4 and grp + 1 < NGRP:
                idx_load(grp + 1, False)
            if j == G - 1 and grp + 1 < NGRP:
                idx_load(grp + 1, True)

    for j in range(2):
        k = NCHUNK - 2 + j
        gwait(k, k % NBUF)
        scatter(k, k % NBUF)
    for j in range(NBUF):
        k = NCHUNK - NBUF + j
        swait(k, k % NBUF)

    plsc.subcore_barrier()
    pltpu.sync_copy(acc.at[pl.ds(base, RPT)],
                    out_hbm.at[pl.ds(cid * NP + base, RPT)])


def _make_conv_kernel():
    mesh = plsc.VectorSubcoreMesh(core_axis_name="c", subcore_axis_name="s")
    return pl.kernel(
        _conv_body,
        out_type=jax.ShapeDtypeStruct((NC * NP, HALF), jnp.float32),
        mesh=mesh,
        compiler_params=pltpu.CompilerParams(needs_layout_passes=False),
        scratch_types=[
            pltpu.VMEM((2, G, CCH), jnp.int32),
            pltpu.VMEM((2, G, CCH), jnp.int32),
            pltpu.VMEM((NBUF, CCH, HALF), jnp.float32),
            pltpu.VMEM_SHARED((NP, HALF), jnp.float32),
            pltpu.SemaphoreType.DMA,
        ] + [pltpu.SemaphoreType.DMA] * (2 * NBUF),
    )


def _split(y):
    return jnp.stack([y[:, :HALF], y[:, HALF:]], axis=0)


def _cat(a):
    return jnp.concatenate([a[0], a[1]], axis=1)


def _prep_body(hist_ref, x_ref, w_ref, dinvb_ref, y_ref):
    deg = jnp.sum(hist_ref[...], axis=1, keepdims=True) + 1.0
    dvb = jnp.broadcast_to(lax.rsqrt(deg), (BR, D))
    dinvb_ref[...] = dvb
    y = jnp.dot(x_ref[...], w_ref[...], preferred_element_type=jnp.float32)
    y_ref[...] = _split(y * dvb)


def _prep_call(hist, x, w1):
    return pl.pallas_call(
        _prep_body,
        grid=(NP // BR,),
        in_specs=[
            pl.BlockSpec((BR, NC * NTILES), lambda i: (i, 0)),
            pl.BlockSpec((BR, D), lambda i: (i, 0)),
            pl.BlockSpec((D, D), lambda i: (0, 0)),
        ],
        out_specs=[
            pl.BlockSpec((BR, D), lambda i: (i, 0)),
            pl.BlockSpec((2, BR, HALF), lambda i: (0, i, 0)),
        ],
        out_shape=[
            jax.ShapeDtypeStruct((NP, D), jnp.float32),
            jax.ShapeDtypeStruct((2, NP, HALF), jnp.float32),
        ],
    )(hist, x, w1)


def _conv_tc_body(relu, nw, acc_ref, dinvb_ref, b_ref, *w_and_out):
    w_refs = w_and_out[:nw]
    out_refs = w_and_out[nw:]
    z = _cat(acc_ref[...])
    dvb = dinvb_ref[...]
    z = z * dvb + b_ref[...]
    if relu:
        z = jnp.maximum(z, 0.0)
    for w_ref, out_ref in zip(w_refs, out_refs):
        y = jnp.dot(z, w_ref[...], preferred_element_type=jnp.float32)
        out_ref[...] = _split(y * dvb)


def _conv_tc_call(acc, dinvb, b, ws, relu):
    nw = len(ws)
    return pl.pallas_call(
        functools.partial(_conv_tc_body, relu, nw),
        grid=(NP // BR,),
        in_specs=[
            pl.BlockSpec((2, BR, HALF), lambda i: (0, i, 0)),
            pl.BlockSpec((BR, D), lambda i: (i, 0)),
            pl.BlockSpec((1, D), lambda i: (0, 0)),
        ] + [pl.BlockSpec((D, D), lambda i: (0, 0))] * nw,
        out_specs=[pl.BlockSpec((2, BR, HALF), lambda i: (0, i, 0))] * nw,
        out_shape=[jax.ShapeDtypeStruct((2, NP, HALF), jnp.float32)] * nw,
    )(acc, dinvb, b, *ws)


def _affine_body(acc_ref, dinvb_ref, b_ref, out_ref):
    out_ref[...] = _cat(acc_ref[...]) * dinvb_ref[...] + b_ref[...]


def _affine_call(acc, dinvb, b):
    return pl.pallas_call(
        _affine_body,
        grid=(NP // BR,),
        in_specs=[
            pl.BlockSpec((2, BR, HALF), lambda i: (0, i, 0)),
            pl.BlockSpec((BR, D), lambda i: (i, 0)),
            pl.BlockSpec((1, D), lambda i: (0, 0)),
        ],
        out_specs=pl.BlockSpec((BR, D), lambda i: (i, 0)),
        out_shape=jax.ShapeDtypeStruct((NP, D), jnp.float32),
    )(acc, dinvb, b)


def _gram_body(a_ref, b_ref, out_ref):
    out_ref[0] = lax.dot_general(
        a_ref[...], b_ref[...], (((1,), (1,)), ((), ())),
        preferred_element_type=jnp.float32)


def _gram_call(h):
    out = pl.pallas_call(
        _gram_body,
        grid=(N // GR, pl.cdiv(N, GC)),
        in_specs=[
            pl.BlockSpec((GR, D), lambda i, j: (i, 0)),
            pl.BlockSpec((GC, D), lambda i, j: (j, 0)),
        ],
        out_specs=pl.BlockSpec((1, GR, GC), lambda i, j: (i, 0, j)),
        out_shape=jax.ShapeDtypeStruct((N // GR, GR, N), jnp.float32),
    )(h, h)
    return out.reshape(N, N)


def _pad_edges(idx, tiles, chunks, width, fill):
    per = chunks * width
    take = E // tiles
    t = idx.reshape(tiles, take)
    pad = jnp.full((tiles, per - take), fill, jnp.int32)
    return jnp.concatenate([t, pad], axis=1).reshape(tiles, chunks, width)


def kernel(x, edge_index, enc_W1, enc_b1, enc_W2, enc_b2,
           attr_W1, attr_b1, attr_W2, attr_b2, str_W1, str_b1):
    src = edge_index[0].astype(jnp.int32)
    dst = edge_index[1].astype(jnp.int32)

    src_p = _pad_edges(src, NTILES, NCHUNK, CCH, 0)
    src_all = jnp.stack([src_p, src_p + NP], axis=0)
    dst_p = _pad_edges(dst, NTILES, NCHUNK, CCH, N)
    dst_d = _pad_edges(dst, NC * NTILES, ED_CH, CH, N)
    x_p = jnp.concatenate([x, jnp.zeros((NP - N, D), jnp.float32)], axis=0)

    deg_k = _make_deg_kernel()
    conv_k = _make_conv_kernel()

    hist = deg_k(dst_d)
    hist_n = hist.reshape(NC * NTILES, NP).T
    dinvb, y1 = _prep_call(hist_n, x_p, enc_W1)

    def conv(y):
        acc = conv_k(y.reshape(NC * NP, HALF), src_all, dst_p)
        return acc.reshape(2, NP, HALF)

    b = lambda v: v.reshape(1, D)

    acc1 = conv(y1)
    (y2,) = _conv_tc_call(acc1, dinvb, b(enc_b1), [enc_W2], relu=True)
    acc2 = conv(y2)
    y3, y5 = _conv_tc_call(acc2, dinvb, b(enc_b2), [attr_W1, str_W1],
                           relu=False)
    acc5 = conv(y5)
    h_full = _affine_call(acc5, dinvb, b(str_b1))
    s_ = _gram_call(h_full[:N])
    acc3 = conv(y3)
    (y4,) = _conv_tc_call(acc3, dinvb, b(attr_b1), [attr_W2], relu=True)
    acc4 = conv(y4)
    x_full = _affine_call(acc4, dinvb, b(attr_b2))
    x_ = x_full[:N]
    return (x_, s_)
